# Initial kernel scaffold; baseline (speedup 1.0000x reference)
import jax
import jax.numpy as jnp
from jax.experimental import pallas as pl


def kernel(coords, batch_inds, features, sdf, occupancy, historical_pose, current_pose):
    raise NotImplementedError("write your pallas kernel here")


# speedup vs baseline: 1.5221x; 1.5221x over previous
import functools

import jax
import jax.numpy as jnp
from jax import lax
from jax.experimental import pallas as pl
from jax.experimental.pallas import tpu as pltpu
from jax.experimental.pallas import tpu_sc as plsc

N = 500000
B = 8
C = 64
R = 800
NCHUNKS = N // R
NW = 32
GROUPS = R // 16

_mesh = plsc.VectorSubcoreMesh(core_axis_name="c", subcore_axis_name="s")


@functools.partial(
    pl.kernel,
    out_type=(
        jax.ShapeDtypeStruct((N * C,), jnp.float32),
        jax.ShapeDtypeStruct((N,), jnp.float32),
        jax.ShapeDtypeStruct((N,), jnp.float32),
        jax.ShapeDtypeStruct((N * 3,), jnp.float32),
        jax.ShapeDtypeStruct((N,), jnp.float32),
    ),
    mesh=_mesh,
    compiler_params=pltpu.CompilerParams(needs_layout_passes=False),
    scratch_types=[
        pltpu.VMEM((B * 16,), jnp.float32),
        pltpu.VMEM((R * 3,), jnp.float32),
        pltpu.VMEM((R,), jnp.int32),
        pltpu.VMEM((R * C,), jnp.float32),
        pltpu.VMEM((R,), jnp.float32),
        pltpu.VMEM((R,), jnp.float32),
        pltpu.VMEM((R,), jnp.float32),
    ],
)
def _pose_project_sc(trans_h, coords_h, binds_h, feats_h, sdf_h, occ_h,
                     ofeat_h, osdf_h, oocc_h, ohc_h, omask_h,
                     trans_v, coords_v, binds_v, feats_v, sdf_v, occ_v, maskf_v):
    wid = lax.axis_index("c") * 16 + lax.axis_index("s")
    pltpu.sync_copy(trans_h, trans_v)
    iota = lax.iota(jnp.int32, 16)
    idx3 = iota * 3
    zeros16 = iota * 0

    nchunks_w = (NCHUNKS - wid + NW - 1) // NW

    def chunk_body(i, carry):
        c = wid + i * NW
        row0 = c * R
        pltpu.sync_copy(coords_h.at[pl.ds(row0 * 3, R * 3)], coords_v)
        pltpu.sync_copy(binds_h.at[pl.ds(row0, R)], binds_v)
        pltpu.sync_copy(feats_h.at[pl.ds(row0 * C, R * C)], feats_v)
        pltpu.sync_copy(sdf_h.at[pl.ds(row0, R)], sdf_v)
        pltpu.sync_copy(occ_h.at[pl.ds(row0, R)], occ_v)

        def group(g, carry2):
            gb = g * 16
            cbase = gb * 3
            b = binds_v[pl.ds(gb, 16)]
            x = plsc.load_gather(coords_v, [cbase + idx3])
            y = plsc.load_gather(coords_v, [cbase + idx3 + 1])
            z = plsc.load_gather(coords_v, [cbase + idx3 + 2])
            tb = b * 16

            def trow(i4):
                t0 = plsc.load_gather(trans_v, [tb + i4])
                t1 = plsc.load_gather(trans_v, [tb + (i4 + 1)])
                t2 = plsc.load_gather(trans_v, [tb + (i4 + 2)])
                t3 = plsc.load_gather(trans_v, [tb + (i4 + 3)])
                return t0 * x + t1 * y + t2 * z + t3

            h0 = trow(0)
            h1 = trow(4)
            h2 = trow(8)
            v0 = h0 * 16.0
            v1 = h1 * 16.0
            v2 = h2 * 16.0
            m = ((v0 >= 0.0) & (v0 < 96.0) & (v1 >= 0.0) & (v1 < 96.0)
                 & (v2 >= 0.0) & (v2 < 48.0))
            mf = jnp.where(m, 1.0, 0.0).astype(jnp.float32)
            maskf_v[pl.ds(gb, 16)] = mf
            plsc.store_scatter(coords_v, [cbase + idx3], h0)
            plsc.store_scatter(coords_v, [cbase + idx3 + 1], h1)
            plsc.store_scatter(coords_v, [cbase + idx3 + 2], h2)
            sdf_v[pl.ds(gb, 16)] = sdf_v[pl.ds(gb, 16)] * mf
            occ_v[pl.ds(gb, 16)] = occ_v[pl.ds(gb, 16)] * mf
            for j in range(16):
                mvj = plsc.load_gather(maskf_v, [zeros16 + (gb + j)])
                fb = (gb + j) * C
                for k in range(C // 16):
                    sl = pl.ds(fb + k * 16, 16)
                    feats_v[sl] = feats_v[sl] * mvj
            return carry2

        lax.fori_loop(0, GROUPS, group, 0)

        pltpu.sync_copy(feats_v, ofeat_h.at[pl.ds(row0 * C, R * C)])
        pltpu.sync_copy(sdf_v, osdf_h.at[pl.ds(row0, R)])
        pltpu.sync_copy(occ_v, oocc_h.at[pl.ds(row0, R)])
        pltpu.sync_copy(coords_v, ohc_h.at[pl.ds(row0 * 3, R * 3)])
        pltpu.sync_copy(maskf_v, omask_h.at[pl.ds(row0, R)])
        return carry

    lax.fori_loop(0, nchunks_w, chunk_body, 0)


def kernel(coords, batch_inds, features, sdf, occupancy, historical_pose, current_pose):
    transform = jnp.einsum("bij,bjk->bik", jnp.linalg.inv(current_pose),
                           historical_pose)
    ofeat, osdf, oocc, ohc, omaskf = _pose_project_sc(
        transform.reshape(B * 16),
        coords.reshape(N * 3),
        batch_inds.astype(jnp.int32),
        features.reshape(N * C),
        sdf.reshape(N),
        occupancy.reshape(N),
    )
    return (ofeat.reshape(N, C), osdf.reshape(N, 1), oocc.reshape(N, 1),
            ohc.reshape(N, 3), omaskf.astype(bool))


# speedup vs baseline: 12.5891x; 8.2710x over previous
import functools

import jax
import jax.numpy as jnp
from jax import lax
from jax.experimental import pallas as pl
from jax.experimental.pallas import tpu as pltpu
from jax.experimental.pallas import tpu_sc as plsc

N = 500000
B = 8
C = 64
R = 2000
NCHUNKS = N // R
NW = 32
GROUPS = R // 16

WN = 2048
GRID = (N + WN - 1) // WN

_mesh = plsc.VectorSubcoreMesh(core_axis_name="c", subcore_axis_name="s")


@functools.partial(
    pl.kernel,
    out_type=(
        jax.ShapeDtypeStruct((N,), jnp.float32),
        jax.ShapeDtypeStruct((N,), jnp.float32),
        jax.ShapeDtypeStruct((N,), jnp.float32),
        jax.ShapeDtypeStruct((N,), jnp.float32),
        jax.ShapeDtypeStruct((N,), jnp.float32),
        jax.ShapeDtypeStruct((N,), jnp.float32),
    ),
    mesh=_mesh,
    compiler_params=pltpu.CompilerParams(needs_layout_passes=False),
    scratch_types=[
        pltpu.VMEM((B * 16,), jnp.float32),
        pltpu.VMEM((R,), jnp.float32),
        pltpu.VMEM((R,), jnp.float32),
        pltpu.VMEM((R,), jnp.float32),
        pltpu.VMEM((R,), jnp.int32),
        pltpu.VMEM((R,), jnp.float32),
        pltpu.VMEM((R,), jnp.float32),
        pltpu.VMEM((R,), jnp.float32),
    ],
)
def _pose_project_sc(trans_h, x_h, y_h, z_h, binds_h, sdf_h, occ_h,
                     ohx_h, ohy_h, ohz_h, omask_h, osdf_h, oocc_h,
                     trans_v, x_v, y_v, z_v, binds_v, sdf_v, occ_v, maskf_v):
    wid = lax.axis_index("c") * 16 + lax.axis_index("s")
    pltpu.sync_copy(trans_h, trans_v)

    nchunks_w = (NCHUNKS - wid + NW - 1) // NW

    def chunk_body(i, carry):
        row0 = (wid + i * NW) * R
        pltpu.sync_copy(x_h.at[pl.ds(row0, R)], x_v)
        pltpu.sync_copy(y_h.at[pl.ds(row0, R)], y_v)
        pltpu.sync_copy(z_h.at[pl.ds(row0, R)], z_v)
        pltpu.sync_copy(binds_h.at[pl.ds(row0, R)], binds_v)
        pltpu.sync_copy(sdf_h.at[pl.ds(row0, R)], sdf_v)
        pltpu.sync_copy(occ_h.at[pl.ds(row0, R)], occ_v)

        def group(g, carry2):
            sl = pl.ds(g * 16, 16)
            b = binds_v[sl]
            x = x_v[sl]
            y = y_v[sl]
            z = z_v[sl]
            tb = b * 16

            def trow(i4):
                t0 = plsc.load_gather(trans_v, [tb + i4])
                t1 = plsc.load_gather(trans_v, [tb + (i4 + 1)])
                t2 = plsc.load_gather(trans_v, [tb + (i4 + 2)])
                t3 = plsc.load_gather(trans_v, [tb + (i4 + 3)])
                return t0 * x + t1 * y + t2 * z + t3

            h0 = trow(0)
            h1 = trow(4)
            h2 = trow(8)
            v0 = h0 * 16.0
            v1 = h1 * 16.0
            v2 = h2 * 16.0
            m = ((v0 >= 0.0) & (v0 < 96.0) & (v1 >= 0.0) & (v1 < 96.0)
                 & (v2 >= 0.0) & (v2 < 48.0))
            mf = jnp.where(m, 1.0, 0.0).astype(jnp.float32)
            x_v[sl] = h0
            y_v[sl] = h1
            z_v[sl] = h2
            maskf_v[sl] = mf
            sdf_v[sl] = sdf_v[sl] * mf
            occ_v[sl] = occ_v[sl] * mf
            return carry2

        lax.fori_loop(0, GROUPS, group, 0)

        pltpu.sync_copy(x_v, ohx_h.at[pl.ds(row0, R)])
        pltpu.sync_copy(y_v, ohy_h.at[pl.ds(row0, R)])
        pltpu.sync_copy(z_v, ohz_h.at[pl.ds(row0, R)])
        pltpu.sync_copy(maskf_v, omask_h.at[pl.ds(row0, R)])
        pltpu.sync_copy(sdf_v, osdf_h.at[pl.ds(row0, R)])
        pltpu.sync_copy(occ_v, oocc_h.at[pl.ds(row0, R)])
        return carry

    lax.fori_loop(0, nchunks_w, chunk_body, 0)


def _feat_body(trans_ref, binds_ref, x_ref, y_ref, z_ref, ft_ref, o_ref):
    b = binds_ref[...]

    def coef(j):
        v = jnp.full((WN,), trans_ref[7, j], jnp.float32)
        for k in range(6, -1, -1):
            v = jnp.where(b == k, trans_ref[k, j], v)
        return v

    x = x_ref[...]
    y = y_ref[...]
    z = z_ref[...]
    h0 = coef(0) * x + coef(1) * y + coef(2) * z + coef(3)
    h1 = coef(4) * x + coef(5) * y + coef(6) * z + coef(7)
    h2 = coef(8) * x + coef(9) * y + coef(10) * z + coef(11)
    v0 = h0 * 16.0
    v1 = h1 * 16.0
    v2 = h2 * 16.0
    m = ((v0 >= 0.0) & (v0 < 96.0) & (v1 >= 0.0) & (v1 < 96.0)
         & (v2 >= 0.0) & (v2 < 48.0))
    mf = jnp.where(m, 1.0, 0.0).astype(jnp.float32)
    o_ref[...] = ft_ref[...] * mf[None, :]


_feat_call = pl.pallas_call(
    _feat_body,
    out_shape=jax.ShapeDtypeStruct((C, N), jnp.float32),
    grid=(GRID,),
    in_specs=[
        pl.BlockSpec(memory_space=pltpu.SMEM),
        pl.BlockSpec((WN,), lambda i: (i,)),
        pl.BlockSpec((WN,), lambda i: (i,)),
        pl.BlockSpec((WN,), lambda i: (i,)),
        pl.BlockSpec((WN,), lambda i: (i,)),
        pl.BlockSpec((C, WN), lambda i: (0, i)),
    ],
    out_specs=pl.BlockSpec((C, WN), lambda i: (0, i)),
    compiler_params=pltpu.CompilerParams(
        dimension_semantics=("arbitrary",)),
)


def kernel(coords, batch_inds, features, sdf, occupancy, historical_pose, current_pose):
    transform = jnp.einsum("bij,bjk->bik", jnp.linalg.inv(current_pose),
                           historical_pose)
    trans_flat = transform.reshape(B * 16)
    binds = batch_inds.astype(jnp.int32)
    x = coords[:, 0]
    y = coords[:, 1]
    z = coords[:, 2]

    hx, hy, hz, maskf, sdfo, occo = _pose_project_sc(
        trans_flat, x, y, z, binds, sdf.reshape(N), occupancy.reshape(N))

    fto = _feat_call(trans_flat.reshape(B, 16), binds, x, y, z, features.T)

    hc = jnp.stack([hx, hy, hz], axis=-1)
    return (fto.T, sdfo.reshape(N, 1), occo.reshape(N, 1), hc,
            maskf.astype(bool))


# speedup vs baseline: 22.0654x; 1.7527x over previous
import functools

import jax
import jax.numpy as jnp
from jax import lax
from jax.experimental import pallas as pl
from jax.experimental.pallas import tpu as pltpu
from jax.experimental.pallas import tpu_sc as plsc

N = 500000
B = 8
C = 64
R = 2000
NCHUNKS = N // R
NW = 32
GROUPS = R // 16

WN = 8192
GRID = (N + WN - 1) // WN

_mesh = plsc.VectorSubcoreMesh(core_axis_name="c", subcore_axis_name="s")


@functools.partial(
    pl.kernel,
    out_type=(
        jax.ShapeDtypeStruct((N,), jnp.float32),
        jax.ShapeDtypeStruct((N,), jnp.float32),
        jax.ShapeDtypeStruct((N,), jnp.float32),
        jax.ShapeDtypeStruct((N,), jnp.float32),
    ),
    mesh=_mesh,
    compiler_params=pltpu.CompilerParams(needs_layout_passes=False),
    scratch_types=[
        pltpu.VMEM((B * 16,), jnp.float32),
        pltpu.VMEM((R,), jnp.float32),
        pltpu.VMEM((R,), jnp.float32),
        pltpu.VMEM((R,), jnp.float32),
        pltpu.VMEM((R,), jnp.int32),
        pltpu.VMEM((R,), jnp.float32),
    ],
)
def _pose_project_sc(trans_h, x_h, y_h, z_h, binds_h,
                     ohx_h, ohy_h, ohz_h, omask_h,
                     trans_v, x_v, y_v, z_v, binds_v, maskf_v):
    wid = lax.axis_index("c") * 16 + lax.axis_index("s")
    pltpu.sync_copy(trans_h, trans_v)

    nchunks_w = (NCHUNKS - wid + NW - 1) // NW

    def chunk_body(i, carry):
        row0 = (wid + i * NW) * R
        pltpu.sync_copy(x_h.at[pl.ds(row0, R)], x_v)
        pltpu.sync_copy(y_h.at[pl.ds(row0, R)], y_v)
        pltpu.sync_copy(z_h.at[pl.ds(row0, R)], z_v)
        pltpu.sync_copy(binds_h.at[pl.ds(row0, R)], binds_v)

        def group(g, carry2):
            sl = pl.ds(g * 16, 16)
            b = binds_v[sl]
            x = x_v[sl]
            y = y_v[sl]
            z = z_v[sl]
            tb = b * 16

            def trow(i4):
                t0 = plsc.load_gather(trans_v, [tb + i4])
                t1 = plsc.load_gather(trans_v, [tb + (i4 + 1)])
                t2 = plsc.load_gather(trans_v, [tb + (i4 + 2)])
                t3 = plsc.load_gather(trans_v, [tb + (i4 + 3)])
                return t0 * x + t1 * y + t2 * z + t3

            h0 = trow(0)
            h1 = trow(4)
            h2 = trow(8)
            v0 = h0 * 16.0
            v1 = h1 * 16.0
            v2 = h2 * 16.0
            m = ((v0 >= 0.0) & (v0 < 96.0) & (v1 >= 0.0) & (v1 < 96.0)
                 & (v2 >= 0.0) & (v2 < 48.0))
            mf = jnp.where(m, 1.0, 0.0).astype(jnp.float32)
            x_v[sl] = h0
            y_v[sl] = h1
            z_v[sl] = h2
            maskf_v[sl] = mf
            return carry2

        lax.fori_loop(0, GROUPS, group, 0)

        pltpu.sync_copy(x_v, ohx_h.at[pl.ds(row0, R)])
        pltpu.sync_copy(y_v, ohy_h.at[pl.ds(row0, R)])
        pltpu.sync_copy(z_v, ohz_h.at[pl.ds(row0, R)])
        pltpu.sync_copy(maskf_v, omask_h.at[pl.ds(row0, R)])
        return carry

    lax.fori_loop(0, nchunks_w, chunk_body, 0)


def _feat_body(trans_ref, binds_ref, x_ref, y_ref, z_ref, ft_ref, o_ref):
    b = binds_ref[...]

    def coef(j):
        v = jnp.full((WN,), trans_ref[7, j], jnp.float32)
        for k in range(6, -1, -1):
            v = jnp.where(b == k, trans_ref[k, j], v)
        return v

    x = x_ref[...]
    y = y_ref[...]
    z = z_ref[...]
    h0 = coef(0) * x + coef(1) * y + coef(2) * z + coef(3)
    h1 = coef(4) * x + coef(5) * y + coef(6) * z + coef(7)
    h2 = coef(8) * x + coef(9) * y + coef(10) * z + coef(11)
    v0 = h0 * 16.0
    v1 = h1 * 16.0
    v2 = h2 * 16.0
    m = ((v0 >= 0.0) & (v0 < 96.0) & (v1 >= 0.0) & (v1 < 96.0)
         & (v2 >= 0.0) & (v2 < 48.0))
    mf = jnp.where(m, 1.0, 0.0).astype(jnp.float32)
    o_ref[...] = ft_ref[...] * mf[None, :]


_feat_call = pl.pallas_call(
    _feat_body,
    out_shape=jax.ShapeDtypeStruct((C, N), jnp.float32),
    grid=(GRID,),
    in_specs=[
        pl.BlockSpec(memory_space=pltpu.SMEM),
        pl.BlockSpec((WN,), lambda i: (i,)),
        pl.BlockSpec((WN,), lambda i: (i,)),
        pl.BlockSpec((WN,), lambda i: (i,)),
        pl.BlockSpec((WN,), lambda i: (i,)),
        pl.BlockSpec((C, WN), lambda i: (0, i)),
    ],
    out_specs=pl.BlockSpec((C, WN), lambda i: (0, i)),
    compiler_params=pltpu.CompilerParams(
        dimension_semantics=("arbitrary",)),
)


def _inv4(m):
    a = [[m[:, i, j] for j in range(4)] for i in range(4)]

    def det3(r, c):
        (i0, i1, i2) = [i for i in range(4) if i != r]
        (j0, j1, j2) = [j for j in range(4) if j != c]
        return (a[i0][j0] * (a[i1][j1] * a[i2][j2] - a[i1][j2] * a[i2][j1])
                - a[i0][j1] * (a[i1][j0] * a[i2][j2] - a[i1][j2] * a[i2][j0])
                + a[i0][j2] * (a[i1][j0] * a[i2][j1] - a[i1][j1] * a[i2][j0]))

    cof = [[((-1.0) ** (i + j)) * det3(i, j) for j in range(4)]
           for i in range(4)]
    det = (a[0][0] * cof[0][0] + a[0][1] * cof[0][1]
           + a[0][2] * cof[0][2] + a[0][3] * cof[0][3])
    inv = jnp.stack([jnp.stack([cof[j][i] for j in range(4)], axis=-1)
                     for i in range(4)], axis=-2)
    return inv / det[:, None, None]


def kernel(coords, batch_inds, features, sdf, occupancy, historical_pose, current_pose):
    transform = jnp.einsum("bij,bjk->bik", _inv4(current_pose),
                           historical_pose)
    trans_flat = transform.reshape(B * 16)
    binds = batch_inds.astype(jnp.int32)
    x = coords[:, 0]
    y = coords[:, 1]
    z = coords[:, 2]

    hx, hy, hz, maskf = _pose_project_sc(trans_flat, x, y, z, binds)

    fto = _feat_call(trans_flat.reshape(B, 16), binds, x, y, z, features.T)

    maskb = maskf.astype(bool)
    proj_sdf = jnp.where(maskb[:, None], sdf, jnp.zeros_like(sdf))
    proj_occ = jnp.where(maskb[:, None], occupancy, jnp.zeros_like(occupancy))
    hc = jnp.stack([hx, hy, hz], axis=-1)
    return (fto.T, proj_sdf, proj_occ, hc, maskb)


# speedup vs baseline: 25.5278x; 1.1569x over previous
import functools

import jax
import jax.numpy as jnp
from jax import lax
from jax.experimental import pallas as pl
from jax.experimental.pallas import tpu as pltpu
from jax.experimental.pallas import tpu_sc as plsc

N = 500000
B = 8
C = 64
R = 2000
NCHUNKS = N // R
NW = 32
GROUPS = R // 16

WN = 8192
GRID = (N + WN - 1) // WN

_mesh = plsc.VectorSubcoreMesh(core_axis_name="c", subcore_axis_name="s")


@functools.partial(
    pl.kernel,
    out_type=(
        jax.ShapeDtypeStruct((N,), jnp.float32),
    ),
    mesh=_mesh,
    compiler_params=pltpu.CompilerParams(needs_layout_passes=False),
    scratch_types=[
        pltpu.VMEM((B * 16,), jnp.float32),
        pltpu.VMEM((R,), jnp.float32),
        pltpu.VMEM((R,), jnp.float32),
        pltpu.VMEM((R,), jnp.float32),
        pltpu.VMEM((R,), jnp.int32),
        pltpu.VMEM((R,), jnp.float32),
    ],
)
def _pose_project_sc(trans_h, x_h, y_h, z_h, binds_h, omask_h,
                     trans_v, x_v, y_v, z_v, binds_v, maskf_v):
    wid = lax.axis_index("c") * 16 + lax.axis_index("s")
    pltpu.sync_copy(trans_h, trans_v)

    nchunks_w = (NCHUNKS - wid + NW - 1) // NW

    def chunk_body(i, carry):
        row0 = (wid + i * NW) * R
        pltpu.sync_copy(x_h.at[pl.ds(row0, R)], x_v)
        pltpu.sync_copy(y_h.at[pl.ds(row0, R)], y_v)
        pltpu.sync_copy(z_h.at[pl.ds(row0, R)], z_v)
        pltpu.sync_copy(binds_h.at[pl.ds(row0, R)], binds_v)

        def group(g, carry2):
            sl = pl.ds(g * 16, 16)
            b = binds_v[sl]
            x = x_v[sl]
            y = y_v[sl]
            z = z_v[sl]
            tb = b * 16

            def trow(i4):
                t0 = plsc.load_gather(trans_v, [tb + i4])
                t1 = plsc.load_gather(trans_v, [tb + (i4 + 1)])
                t2 = plsc.load_gather(trans_v, [tb + (i4 + 2)])
                t3 = plsc.load_gather(trans_v, [tb + (i4 + 3)])
                return t0 * x + t1 * y + t2 * z + t3

            h0 = trow(0)
            h1 = trow(4)
            h2 = trow(8)
            v0 = h0 * 16.0
            v1 = h1 * 16.0
            v2 = h2 * 16.0
            m = ((v0 >= 0.0) & (v0 < 96.0) & (v1 >= 0.0) & (v1 < 96.0)
                 & (v2 >= 0.0) & (v2 < 48.0))
            mf = jnp.where(m, 1.0, 0.0).astype(jnp.float32)
            maskf_v[sl] = mf
            return carry2

        lax.fori_loop(0, GROUPS, group, 0)

        pltpu.sync_copy(maskf_v, omask_h.at[pl.ds(row0, R)])
        return carry

    lax.fori_loop(0, nchunks_w, chunk_body, 0)


def _feat_body(trans_ref, binds_ref, x_ref, y_ref, z_ref, ft_ref, o_ref, hc_ref):
    b = binds_ref[...]

    def coef(j):
        v = jnp.full((WN,), trans_ref[7, j], jnp.float32)
        for k in range(6, -1, -1):
            v = jnp.where(b == k, trans_ref[k, j], v)
        return v

    x = x_ref[...]
    y = y_ref[...]
    z = z_ref[...]
    h0 = coef(0) * x + coef(1) * y + coef(2) * z + coef(3)
    h1 = coef(4) * x + coef(5) * y + coef(6) * z + coef(7)
    h2 = coef(8) * x + coef(9) * y + coef(10) * z + coef(11)
    v0 = h0 * 16.0
    v1 = h1 * 16.0
    v2 = h2 * 16.0
    m = ((v0 >= 0.0) & (v0 < 96.0) & (v1 >= 0.0) & (v1 < 96.0)
         & (v2 >= 0.0) & (v2 < 48.0))
    mf = jnp.where(m, 1.0, 0.0).astype(jnp.float32)
    o_ref[...] = ft_ref[...] * mf[None, :]
    hc_ref[0:1, :] = h0[None, :]
    hc_ref[1:2, :] = h1[None, :]
    hc_ref[2:3, :] = h2[None, :]
    hc_ref[3:4, :] = mf[None, :]


_feat_call = pl.pallas_call(
    _feat_body,
    out_shape=(
        jax.ShapeDtypeStruct((C, N), jnp.float32),
        jax.ShapeDtypeStruct((4, N), jnp.float32),
    ),
    grid=(GRID,),
    in_specs=[
        pl.BlockSpec(memory_space=pltpu.SMEM),
        pl.BlockSpec((WN,), lambda i: (i,)),
        pl.BlockSpec((WN,), lambda i: (i,)),
        pl.BlockSpec((WN,), lambda i: (i,)),
        pl.BlockSpec((WN,), lambda i: (i,)),
        pl.BlockSpec((C, WN), lambda i: (0, i)),
    ],
    out_specs=(
        pl.BlockSpec((C, WN), lambda i: (0, i)),
        pl.BlockSpec((4, WN), lambda i: (0, i)),
    ),
    compiler_params=pltpu.CompilerParams(
        dimension_semantics=("arbitrary",)),
)


def _inv4(m):
    a = [[m[:, i, j] for j in range(4)] for i in range(4)]

    def det3(r, c):
        (i0, i1, i2) = [i for i in range(4) if i != r]
        (j0, j1, j2) = [j for j in range(4) if j != c]
        return (a[i0][j0] * (a[i1][j1] * a[i2][j2] - a[i1][j2] * a[i2][j1])
                - a[i0][j1] * (a[i1][j0] * a[i2][j2] - a[i1][j2] * a[i2][j0])
                + a[i0][j2] * (a[i1][j0] * a[i2][j1] - a[i1][j1] * a[i2][j0]))

    cof = [[((-1.0) ** (i + j)) * det3(i, j) for j in range(4)]
           for i in range(4)]
    det = (a[0][0] * cof[0][0] + a[0][1] * cof[0][1]
           + a[0][2] * cof[0][2] + a[0][3] * cof[0][3])
    inv = jnp.stack([jnp.stack([cof[j][i] for j in range(4)], axis=-1)
                     for i in range(4)], axis=-2)
    return inv / det[:, None, None]


def kernel(coords, batch_inds, features, sdf, occupancy, historical_pose, current_pose):
    transform = jnp.einsum("bij,bjk->bik", _inv4(current_pose),
                           historical_pose)
    trans_flat = transform.reshape(B * 16)
    binds = batch_inds.astype(jnp.int32)
    x = coords[:, 0]
    y = coords[:, 1]
    z = coords[:, 2]

    (maskf,) = _pose_project_sc(trans_flat, x, y, z, binds)

    fto, hc4 = _feat_call(trans_flat.reshape(B, 16), binds, x, y, z,
                          features.T)

    maskb = maskf.astype(bool)
    proj_sdf = jnp.where(maskb[:, None], sdf, jnp.zeros_like(sdf))
    proj_occ = jnp.where(maskb[:, None], occupancy, jnp.zeros_like(occupancy))
    hc = hc4[:3].T
    return (fto.T, proj_sdf, proj_occ, hc, maskb)


# speedup vs baseline: 25.8368x; 1.0121x over previous
import functools

import jax
import jax.numpy as jnp
from jax import lax
from jax.experimental import pallas as pl
from jax.experimental.pallas import tpu as pltpu
from jax.experimental.pallas import tpu_sc as plsc

N = 500000
B = 8
C = 64
R = 2000
NCHUNKS = N // R
NW = 32
GROUPS = R // 16

WN = 16384
GRID = (N + WN - 1) // WN

_mesh = plsc.VectorSubcoreMesh(core_axis_name="c", subcore_axis_name="s")


@functools.partial(
    pl.kernel,
    out_type=(
        jax.ShapeDtypeStruct((N,), jnp.float32),
    ),
    mesh=_mesh,
    compiler_params=pltpu.CompilerParams(needs_layout_passes=False),
    scratch_types=[
        pltpu.VMEM((B * 16,), jnp.float32),
        pltpu.VMEM((R,), jnp.float32),
        pltpu.VMEM((R,), jnp.float32),
        pltpu.VMEM((R,), jnp.float32),
        pltpu.VMEM((R,), jnp.int32),
        pltpu.VMEM((R,), jnp.float32),
    ],
)
def _pose_project_sc(trans_h, x_h, y_h, z_h, binds_h, omask_h,
                     trans_v, x_v, y_v, z_v, binds_v, maskf_v):
    wid = lax.axis_index("c") * 16 + lax.axis_index("s")
    pltpu.sync_copy(trans_h, trans_v)

    nchunks_w = (NCHUNKS - wid + NW - 1) // NW

    def chunk_body(i, carry):
        row0 = (wid + i * NW) * R
        pltpu.sync_copy(x_h.at[pl.ds(row0, R)], x_v)
        pltpu.sync_copy(y_h.at[pl.ds(row0, R)], y_v)
        pltpu.sync_copy(z_h.at[pl.ds(row0, R)], z_v)
        pltpu.sync_copy(binds_h.at[pl.ds(row0, R)], binds_v)

        def group(g, carry2):
            sl = pl.ds(g * 16, 16)
            b = binds_v[sl]
            x = x_v[sl]
            y = y_v[sl]
            z = z_v[sl]
            tb = b * 16

            def trow(i4):
                t0 = plsc.load_gather(trans_v, [tb + i4])
                t1 = plsc.load_gather(trans_v, [tb + (i4 + 1)])
                t2 = plsc.load_gather(trans_v, [tb + (i4 + 2)])
                t3 = plsc.load_gather(trans_v, [tb + (i4 + 3)])
                return t0 * x + t1 * y + t2 * z + t3

            h0 = trow(0)
            h1 = trow(4)
            h2 = trow(8)
            v0 = h0 * 16.0
            v1 = h1 * 16.0
            v2 = h2 * 16.0
            m = ((v0 >= 0.0) & (v0 < 96.0) & (v1 >= 0.0) & (v1 < 96.0)
                 & (v2 >= 0.0) & (v2 < 48.0))
            mf = jnp.where(m, 1.0, 0.0).astype(jnp.float32)
            maskf_v[sl] = mf
            return carry2

        lax.fori_loop(0, GROUPS, group, 0, unroll=4)

        pltpu.sync_copy(maskf_v, omask_h.at[pl.ds(row0, R)])
        return carry

    lax.fori_loop(0, nchunks_w, chunk_body, 0)


def _feat_body(trans_ref, binds_ref, x_ref, y_ref, z_ref, ft_ref, o_ref, hc_ref):
    b = binds_ref[...]

    def coef(j):
        v = jnp.full((WN,), trans_ref[7, j], jnp.float32)
        for k in range(6, -1, -1):
            v = jnp.where(b == k, trans_ref[k, j], v)
        return v

    x = x_ref[...]
    y = y_ref[...]
    z = z_ref[...]
    h0 = coef(0) * x + coef(1) * y + coef(2) * z + coef(3)
    h1 = coef(4) * x + coef(5) * y + coef(6) * z + coef(7)
    h2 = coef(8) * x + coef(9) * y + coef(10) * z + coef(11)
    v0 = h0 * 16.0
    v1 = h1 * 16.0
    v2 = h2 * 16.0
    m = ((v0 >= 0.0) & (v0 < 96.0) & (v1 >= 0.0) & (v1 < 96.0)
         & (v2 >= 0.0) & (v2 < 48.0))
    mf = jnp.where(m, 1.0, 0.0).astype(jnp.float32)
    o_ref[...] = ft_ref[...] * mf[None, :]
    hc_ref[0:1, :] = h0[None, :]
    hc_ref[1:2, :] = h1[None, :]
    hc_ref[2:3, :] = h2[None, :]
    hc_ref[3:4, :] = mf[None, :]


_feat_call = pl.pallas_call(
    _feat_body,
    out_shape=(
        jax.ShapeDtypeStruct((C, N), jnp.float32),
        jax.ShapeDtypeStruct((4, N), jnp.float32),
    ),
    grid=(GRID,),
    in_specs=[
        pl.BlockSpec(memory_space=pltpu.SMEM),
        pl.BlockSpec((WN,), lambda i: (i,)),
        pl.BlockSpec((WN,), lambda i: (i,)),
        pl.BlockSpec((WN,), lambda i: (i,)),
        pl.BlockSpec((WN,), lambda i: (i,)),
        pl.BlockSpec((C, WN), lambda i: (0, i)),
    ],
    out_specs=(
        pl.BlockSpec((C, WN), lambda i: (0, i)),
        pl.BlockSpec((4, WN), lambda i: (0, i)),
    ),
    compiler_params=pltpu.CompilerParams(
        dimension_semantics=("arbitrary",)),
)


def _inv4(m):
    a = [[m[:, i, j] for j in range(4)] for i in range(4)]

    def det3(r, c):
        (i0, i1, i2) = [i for i in range(4) if i != r]
        (j0, j1, j2) = [j for j in range(4) if j != c]
        return (a[i0][j0] * (a[i1][j1] * a[i2][j2] - a[i1][j2] * a[i2][j1])
                - a[i0][j1] * (a[i1][j0] * a[i2][j2] - a[i1][j2] * a[i2][j0])
                + a[i0][j2] * (a[i1][j0] * a[i2][j1] - a[i1][j1] * a[i2][j0]))

    cof = [[((-1.0) ** (i + j)) * det3(i, j) for j in range(4)]
           for i in range(4)]
    det = (a[0][0] * cof[0][0] + a[0][1] * cof[0][1]
           + a[0][2] * cof[0][2] + a[0][3] * cof[0][3])
    inv = jnp.stack([jnp.stack([cof[j][i] for j in range(4)], axis=-1)
                     for i in range(4)], axis=-2)
    return inv / det[:, None, None]


def kernel(coords, batch_inds, features, sdf, occupancy, historical_pose, current_pose):
    transform = jnp.einsum("bij,bjk->bik", _inv4(current_pose),
                           historical_pose)
    trans_flat = transform.reshape(B * 16)
    binds = batch_inds.astype(jnp.int32)
    x = coords[:, 0]
    y = coords[:, 1]
    z = coords[:, 2]

    (maskf,) = _pose_project_sc(trans_flat, x, y, z, binds)

    fto, hc4 = _feat_call(trans_flat.reshape(B, 16), binds, x, y, z,
                          features.T)

    maskb = maskf.astype(bool)
    proj_sdf = jnp.where(maskb[:, None], sdf, jnp.zeros_like(sdf))
    proj_occ = jnp.where(maskb[:, None], occupancy, jnp.zeros_like(occupancy))
    hc = hc4[:3].T
    return (fto.T, proj_sdf, proj_occ, hc, maskb)


# speedup vs baseline: 26.6772x; 1.0325x over previous
import functools

import jax
import jax.numpy as jnp
from jax import lax
from jax.experimental import pallas as pl
from jax.experimental.pallas import tpu as pltpu
from jax.experimental.pallas import tpu_sc as plsc

N = 500000
B = 8
C = 64
R = 2000
NCHUNKS = N // R
NW = 32
GROUPS = R // 16

WN = 16384
GRID = (N + WN - 1) // WN

_mesh = plsc.VectorSubcoreMesh(core_axis_name="c", subcore_axis_name="s")


@functools.partial(
    pl.kernel,
    out_type=(
        jax.ShapeDtypeStruct((N,), jnp.float32),
    ),
    mesh=_mesh,
    compiler_params=pltpu.CompilerParams(needs_layout_passes=False),
    scratch_types=[
        pltpu.VMEM((B * 16,), jnp.float32),
        pltpu.VMEM((R,), jnp.float32),
        pltpu.VMEM((R,), jnp.float32),
        pltpu.VMEM((R,), jnp.float32),
        pltpu.VMEM((R,), jnp.float32),
        pltpu.VMEM((R,), jnp.float32),
        pltpu.VMEM((R,), jnp.float32),
        pltpu.VMEM((R,), jnp.int32),
        pltpu.VMEM((R,), jnp.int32),
        pltpu.VMEM((R,), jnp.float32),
        pltpu.VMEM((R,), jnp.float32),
        pltpu.SemaphoreType.DMA((2,)),
    ],
)
def _pose_project_sc(trans_h, x_h, y_h, z_h, binds_h, omask_h,
                     trans_v, x_v0, x_v1, y_v0, y_v1, z_v0, z_v1,
                     b_v0, b_v1, m_v0, m_v1, sem_in):
    wid = lax.axis_index("c") * 16 + lax.axis_index("s")
    pltpu.sync_copy(trans_h, trans_v)
    xs = (x_v0, x_v1)
    ys = (y_v0, y_v1)
    zs = (z_v0, z_v1)
    bs = (b_v0, b_v1)
    ms = (m_v0, m_v1)

    nchunks_w = (NCHUNKS - wid + NW - 1) // NW

    def start_in(c, s):
        row0 = (wid + c * NW) * R
        sl = pl.ds(row0, R)
        pltpu.async_copy(x_h.at[sl], xs[s], sem_in.at[s])
        pltpu.async_copy(y_h.at[sl], ys[s], sem_in.at[s])
        pltpu.async_copy(z_h.at[sl], zs[s], sem_in.at[s])
        pltpu.async_copy(binds_h.at[sl], bs[s], sem_in.at[s])

    def wait_in(s):
        pltpu.make_async_copy(x_h.at[pl.ds(0, R)], xs[s], sem_in.at[s]).wait()
        pltpu.make_async_copy(y_h.at[pl.ds(0, R)], ys[s], sem_in.at[s]).wait()
        pltpu.make_async_copy(z_h.at[pl.ds(0, R)], zs[s], sem_in.at[s]).wait()
        pltpu.make_async_copy(binds_h.at[pl.ds(0, R)], bs[s],
                              sem_in.at[s]).wait()

    def compute(c, s):
        def group(g, carry2):
            sl = pl.ds(g * 16, 16)
            b = bs[s][sl]
            x = xs[s][sl]
            y = ys[s][sl]
            z = zs[s][sl]
            tb = b * 16

            def trow(i4):
                t0 = plsc.load_gather(trans_v, [tb + i4])
                t1 = plsc.load_gather(trans_v, [tb + (i4 + 1)])
                t2 = plsc.load_gather(trans_v, [tb + (i4 + 2)])
                t3 = plsc.load_gather(trans_v, [tb + (i4 + 3)])
                return t0 * x + t1 * y + t2 * z + t3

            h0 = trow(0)
            h1 = trow(4)
            h2 = trow(8)
            v0 = h0 * 16.0
            v1 = h1 * 16.0
            v2 = h2 * 16.0
            m = ((v0 >= 0.0) & (v0 < 96.0) & (v1 >= 0.0) & (v1 < 96.0)
                 & (v2 >= 0.0) & (v2 < 48.0))
            mf = jnp.where(m, 1.0, 0.0).astype(jnp.float32)
            ms[s][sl] = mf
            return carry2

        lax.fori_loop(0, GROUPS, group, 0)
        row0 = (wid + c * NW) * R
        pltpu.sync_copy(ms[s], omask_h.at[pl.ds(row0, R)])

    @pl.when(nchunks_w > 0)
    def _prologue():
        start_in(0, 0)

    def chunk_pair(i, carry):
        c0 = i * 2
        c1 = c0 + 1

        @pl.when(c1 < nchunks_w)
        def _s1():
            start_in(c1, 1)

        wait_in(0)
        compute(c0, 0)

        @pl.when(c1 < nchunks_w)
        def _s1b():
            @pl.when(c1 + 1 < nchunks_w)
            def _s0n():
                start_in(c1 + 1, 0)

            wait_in(1)
            compute(c1, 1)

        return carry

    lax.fori_loop(0, (nchunks_w + 1) // 2, chunk_pair, 0)


def _feat_body(trans_ref, binds_ref, x_ref, y_ref, z_ref, ft_ref, o_ref, hc_ref):
    b = binds_ref[...]

    def coef(j):
        v = jnp.full((WN,), trans_ref[7, j], jnp.float32)
        for k in range(6, -1, -1):
            v = jnp.where(b == k, trans_ref[k, j], v)
        return v

    x = x_ref[...]
    y = y_ref[...]
    z = z_ref[...]
    h0 = coef(0) * x + coef(1) * y + coef(2) * z + coef(3)
    h1 = coef(4) * x + coef(5) * y + coef(6) * z + coef(7)
    h2 = coef(8) * x + coef(9) * y + coef(10) * z + coef(11)
    v0 = h0 * 16.0
    v1 = h1 * 16.0
    v2 = h2 * 16.0
    m = ((v0 >= 0.0) & (v0 < 96.0) & (v1 >= 0.0) & (v1 < 96.0)
         & (v2 >= 0.0) & (v2 < 48.0))
    mf = jnp.where(m, 1.0, 0.0).astype(jnp.float32)
    o_ref[...] = ft_ref[...] * mf[None, :]
    hc_ref[0:1, :] = h0[None, :]
    hc_ref[1:2, :] = h1[None, :]
    hc_ref[2:3, :] = h2[None, :]
    hc_ref[3:4, :] = mf[None, :]


_feat_call = pl.pallas_call(
    _feat_body,
    out_shape=(
        jax.ShapeDtypeStruct((C, N), jnp.float32),
        jax.ShapeDtypeStruct((4, N), jnp.float32),
    ),
    grid=(GRID,),
    in_specs=[
        pl.BlockSpec(memory_space=pltpu.SMEM),
        pl.BlockSpec((WN,), lambda i: (i,)),
        pl.BlockSpec((WN,), lambda i: (i,)),
        pl.BlockSpec((WN,), lambda i: (i,)),
        pl.BlockSpec((WN,), lambda i: (i,)),
        pl.BlockSpec((C, WN), lambda i: (0, i)),
    ],
    out_specs=(
        pl.BlockSpec((C, WN), lambda i: (0, i)),
        pl.BlockSpec((4, WN), lambda i: (0, i)),
    ),
    compiler_params=pltpu.CompilerParams(
        dimension_semantics=("arbitrary",)),
)


def _inv4(m):
    a = [[m[:, i, j] for j in range(4)] for i in range(4)]

    def det3(r, c):
        (i0, i1, i2) = [i for i in range(4) if i != r]
        (j0, j1, j2) = [j for j in range(4) if j != c]
        return (a[i0][j0] * (a[i1][j1] * a[i2][j2] - a[i1][j2] * a[i2][j1])
                - a[i0][j1] * (a[i1][j0] * a[i2][j2] - a[i1][j2] * a[i2][j0])
                + a[i0][j2] * (a[i1][j0] * a[i2][j1] - a[i1][j1] * a[i2][j0]))

    cof = [[((-1.0) ** (i + j)) * det3(i, j) for j in range(4)]
           for i in range(4)]
    det = (a[0][0] * cof[0][0] + a[0][1] * cof[0][1]
           + a[0][2] * cof[0][2] + a[0][3] * cof[0][3])
    inv = jnp.stack([jnp.stack([cof[j][i] for j in range(4)], axis=-1)
                     for i in range(4)], axis=-2)
    return inv / det[:, None, None]


def kernel(coords, batch_inds, features, sdf, occupancy, historical_pose, current_pose):
    transform = jnp.einsum("bij,bjk->bik", _inv4(current_pose),
                           historical_pose)
    trans_flat = transform.reshape(B * 16)
    binds = batch_inds.astype(jnp.int32)
    x = coords[:, 0]
    y = coords[:, 1]
    z = coords[:, 2]

    (maskf,) = _pose_project_sc(trans_flat, x, y, z, binds)

    fto, hc4 = _feat_call(trans_flat.reshape(B, 16), binds, x, y, z,
                          features.T)

    maskb = maskf.astype(bool)
    proj_sdf = jnp.where(maskb[:, None], sdf, jnp.zeros_like(sdf))
    proj_occ = jnp.where(maskb[:, None], occupancy, jnp.zeros_like(occupancy))
    hc = hc4[:3].T
    return (fto.T, proj_sdf, proj_occ, hc, maskb)


# speedup vs baseline: 27.9065x; 1.0461x over previous
import functools

import jax
import jax.numpy as jnp
from jax import lax
from jax.experimental import pallas as pl
from jax.experimental.pallas import tpu as pltpu
from jax.experimental.pallas import tpu_sc as plsc

N = 500000
B = 8
C = 64
R = 2000
NCHUNKS = N // R
NW = 32
GROUPS = R // 16

WN = 16384
GRID = (N + WN - 1) // WN

_mesh = plsc.VectorSubcoreMesh(core_axis_name="c", subcore_axis_name="s")


@functools.partial(
    pl.kernel,
    out_type=(
        jax.ShapeDtypeStruct((N,), jnp.float32),
    ),
    mesh=_mesh,
    compiler_params=pltpu.CompilerParams(needs_layout_passes=False),
    scratch_types=[
        pltpu.VMEM((B * 16,), jnp.float32),
        pltpu.VMEM((R,), jnp.float32),
        pltpu.VMEM((R,), jnp.float32),
        pltpu.VMEM((R,), jnp.float32),
        pltpu.VMEM((R,), jnp.float32),
        pltpu.VMEM((R,), jnp.float32),
        pltpu.VMEM((R,), jnp.float32),
        pltpu.VMEM((R,), jnp.int32),
        pltpu.VMEM((R,), jnp.int32),
        pltpu.VMEM((R,), jnp.float32),
        pltpu.VMEM((R,), jnp.float32),
        pltpu.SemaphoreType.DMA((2,)),
    ],
)
def _pose_project_sc(trans_h, x_h, y_h, z_h, binds_h, omask_h,
                     trans_v, x_v0, x_v1, y_v0, y_v1, z_v0, z_v1,
                     b_v0, b_v1, m_v0, m_v1, sem_in):
    wid = lax.axis_index("c") * 16 + lax.axis_index("s")
    pltpu.sync_copy(trans_h, trans_v)
    xs = (x_v0, x_v1)
    ys = (y_v0, y_v1)
    zs = (z_v0, z_v1)
    bs = (b_v0, b_v1)
    ms = (m_v0, m_v1)

    nchunks_w = (NCHUNKS - wid + NW - 1) // NW

    def start_in(c, s):
        row0 = (wid + c * NW) * R
        sl = pl.ds(row0, R)
        pltpu.async_copy(x_h.at[sl], xs[s], sem_in.at[s])
        pltpu.async_copy(y_h.at[sl], ys[s], sem_in.at[s])
        pltpu.async_copy(z_h.at[sl], zs[s], sem_in.at[s])
        pltpu.async_copy(binds_h.at[sl], bs[s], sem_in.at[s])

    def wait_in(s):
        pltpu.make_async_copy(x_h.at[pl.ds(0, R)], xs[s], sem_in.at[s]).wait()
        pltpu.make_async_copy(y_h.at[pl.ds(0, R)], ys[s], sem_in.at[s]).wait()
        pltpu.make_async_copy(z_h.at[pl.ds(0, R)], zs[s], sem_in.at[s]).wait()
        pltpu.make_async_copy(binds_h.at[pl.ds(0, R)], bs[s],
                              sem_in.at[s]).wait()

    def compute(c, s):
        def group(g, carry2):
            sl = pl.ds(g * 16, 16)
            b = bs[s][sl]
            x = xs[s][sl]
            y = ys[s][sl]
            z = zs[s][sl]
            tb = b * 16

            def trow(i4):
                t0 = plsc.load_gather(trans_v, [tb + i4])
                t1 = plsc.load_gather(trans_v, [tb + (i4 + 1)])
                t2 = plsc.load_gather(trans_v, [tb + (i4 + 2)])
                t3 = plsc.load_gather(trans_v, [tb + (i4 + 3)])
                return t0 * x + t1 * y + t2 * z + t3

            h0 = trow(0)
            h1 = trow(4)
            h2 = trow(8)
            v0 = h0 * 16.0
            v1 = h1 * 16.0
            v2 = h2 * 16.0
            m = ((v0 >= 0.0) & (v0 < 96.0) & (v1 >= 0.0) & (v1 < 96.0)
                 & (v2 >= 0.0) & (v2 < 48.0))
            mf = jnp.where(m, 1.0, 0.0).astype(jnp.float32)
            ms[s][sl] = mf
            return carry2

        lax.fori_loop(0, GROUPS, group, 0)
        row0 = (wid + c * NW) * R
        pltpu.sync_copy(ms[s], omask_h.at[pl.ds(row0, R)])

    @pl.when(nchunks_w > 0)
    def _prologue():
        start_in(0, 0)

    def chunk_pair(i, carry):
        c0 = i * 2
        c1 = c0 + 1

        @pl.when(c1 < nchunks_w)
        def _s1():
            start_in(c1, 1)

        wait_in(0)
        compute(c0, 0)

        @pl.when(c1 < nchunks_w)
        def _s1b():
            @pl.when(c1 + 1 < nchunks_w)
            def _s0n():
                start_in(c1 + 1, 0)

            wait_in(1)
            compute(c1, 1)

        return carry

    lax.fori_loop(0, (nchunks_w + 1) // 2, chunk_pair, 0)


def _feat_body(trans_ref, binds_ref, x_ref, y_ref, z_ref, ft_ref, o_ref, hc_ref):
    b = binds_ref[...]

    def coef(j):
        v = jnp.full((WN,), trans_ref[7, j], jnp.float32)
        for k in range(6, -1, -1):
            v = jnp.where(b == k, trans_ref[k, j], v)
        return v

    x = x_ref[...]
    y = y_ref[...]
    z = z_ref[...]
    h0 = coef(0) * x + coef(1) * y + coef(2) * z + coef(3)
    h1 = coef(4) * x + coef(5) * y + coef(6) * z + coef(7)
    h2 = coef(8) * x + coef(9) * y + coef(10) * z + coef(11)
    v0 = h0 * 16.0
    v1 = h1 * 16.0
    v2 = h2 * 16.0
    m = ((v0 >= 0.0) & (v0 < 96.0) & (v1 >= 0.0) & (v1 < 96.0)
         & (v2 >= 0.0) & (v2 < 48.0))
    mf = jnp.where(m, 1.0, 0.0).astype(jnp.float32)
    o_ref[...] = ft_ref[...] * mf[None, :]
    hc_ref[0:1, :] = h0[None, :]
    hc_ref[1:2, :] = h1[None, :]
    hc_ref[2:3, :] = h2[None, :]


_feat_call = pl.pallas_call(
    _feat_body,
    out_shape=(
        jax.ShapeDtypeStruct((C, N), jnp.float32),
        jax.ShapeDtypeStruct((3, N), jnp.float32),
    ),
    grid=(GRID,),
    in_specs=[
        pl.BlockSpec(memory_space=pltpu.SMEM),
        pl.BlockSpec((WN,), lambda i: (i,)),
        pl.BlockSpec((WN,), lambda i: (i,)),
        pl.BlockSpec((WN,), lambda i: (i,)),
        pl.BlockSpec((WN,), lambda i: (i,)),
        pl.BlockSpec((C, WN), lambda i: (0, i)),
    ],
    out_specs=(
        pl.BlockSpec((C, WN), lambda i: (0, i)),
        pl.BlockSpec((3, WN), lambda i: (0, i)),
    ),
    compiler_params=pltpu.CompilerParams(
        dimension_semantics=("arbitrary",)),
)


def _inv4(m):
    a = [[m[:, i, j] for j in range(4)] for i in range(4)]

    def det3(r, c):
        (i0, i1, i2) = [i for i in range(4) if i != r]
        (j0, j1, j2) = [j for j in range(4) if j != c]
        return (a[i0][j0] * (a[i1][j1] * a[i2][j2] - a[i1][j2] * a[i2][j1])
                - a[i0][j1] * (a[i1][j0] * a[i2][j2] - a[i1][j2] * a[i2][j0])
                + a[i0][j2] * (a[i1][j0] * a[i2][j1] - a[i1][j1] * a[i2][j0]))

    cof = [[((-1.0) ** (i + j)) * det3(i, j) for j in range(4)]
           for i in range(4)]
    det = (a[0][0] * cof[0][0] + a[0][1] * cof[0][1]
           + a[0][2] * cof[0][2] + a[0][3] * cof[0][3])
    inv = jnp.stack([jnp.stack([cof[j][i] for j in range(4)], axis=-1)
                     for i in range(4)], axis=-2)
    return inv / det[:, None, None]


def kernel(coords, batch_inds, features, sdf, occupancy, historical_pose, current_pose):
    transform = jnp.einsum("bij,bjk->bik", _inv4(current_pose),
                           historical_pose)
    trans_flat = transform.reshape(B * 16)
    binds = batch_inds.astype(jnp.int32)
    x = coords[:, 0]
    y = coords[:, 1]
    z = coords[:, 2]

    (maskf,) = _pose_project_sc(trans_flat, x, y, z, binds)

    fto, hc4 = _feat_call(trans_flat.reshape(B, 16), binds, x, y, z,
                          features.T)

    maskb = maskf.astype(bool)
    proj_sdf = jnp.where(maskb[:, None], sdf, jnp.zeros_like(sdf))
    proj_occ = jnp.where(maskb[:, None], occupancy, jnp.zeros_like(occupancy))
    hc = hc4.T
    return (fto.T, proj_sdf, proj_occ, hc, maskb)
